# baseline (speedup 1.0000x reference)
import functools

import jax
import jax.numpy as jnp
from jax import lax
from jax.experimental import pallas as pl
from jax.experimental.pallas import tpu as pltpu
from jax.experimental.pallas import tpu_sc as plsc

D = 768
LANES = 16
NC = 2
NS = 16
NW = NC * NS
CH = 64


def _embed_sc(x_flat, W, pos_embd, seq_len):
    B = x_flat.shape[0]
    b_per_w = B // NW
    n_chunks = b_per_w // CH
    mesh = plsc.VectorSubcoreMesh(core_axis_name="c", subcore_axis_name="s")

    @functools.partial(
        pl.kernel,
        out_type=jax.ShapeDtypeStruct((B, D), jnp.float32),
        mesh=mesh,
        scratch_types=[
            pltpu.VMEM((CH,), jnp.int32),
            pltpu.VMEM((CH, D), jnp.float32),
            pltpu.VMEM((CH, D), jnp.float32),
            pltpu.SemaphoreType.DMA,
        ],
    )
    def k(x_hbm, w_hbm, pos_hbm, out_hbm, idx_v, rows_v, pos_v, sem):
        wid = lax.axis_index("s") * NC + lax.axis_index("c")
        base = wid * b_per_w
        pos_base = lax.rem(base, seq_len)

        for c in range(n_chunks):
            row0 = base + c * CH
            pltpu.sync_copy(x_hbm.at[pl.ds(row0, CH)], idx_v)
            pltpu.async_copy(w_hbm.at[idx_v], rows_v, sem).wait()
            pltpu.sync_copy(pos_hbm.at[pl.ds(pos_base + c * CH, CH)], pos_v)

            def add_row(r, carry):
                for j in range(D // LANES):
                    sl = pl.ds(j * LANES, LANES)
                    rows_v[r, sl] = rows_v[r, sl] + pos_v[r, sl]
                return carry

            lax.fori_loop(0, CH, add_row, 0)
            pltpu.sync_copy(rows_v, out_hbm.at[pl.ds(row0, CH)])

    return k(x_flat, W, pos_embd)


def kernel(x, W, pos_embd):
    batch, seq_len = x.shape
    x_flat = x.reshape(-1).astype(jnp.int32)
    out = _embed_sc(x_flat, W, pos_embd[:seq_len], seq_len)
    return out.reshape(batch, seq_len, D)
